# Initial kernel scaffold; baseline (speedup 1.0000x reference)
import jax
import jax.numpy as jnp
from jax.experimental import pallas as pl


def kernel(x, edge_index, edge_attr, edge_label_index, target_edge_attr, params):
    raise NotImplementedError("write your pallas kernel here")


# speedup vs baseline: 1.0374x; 1.0374x over previous
import functools

import jax
import jax.numpy as jnp
from jax.experimental import pallas as pl
from jax.experimental.pallas import tpu as pltpu

H = 66
P = 80


def _pad_w(Wb, pin, pout):
    W, b = Wb["W"], Wb["b"]
    Wp = jnp.zeros((pin, pout), jnp.float32).at[: W.shape[0], : W.shape[1]].set(W)
    bp = jnp.zeros((1, pout), jnp.float32).at[0, : b.shape[0]].set(b)
    return Wp, bp


def _linear_body(x_ref, w_ref, b_ref, o_ref):
    o_ref[:, :] = (
        jnp.dot(x_ref[:, :], w_ref[:, :], preferred_element_type=jnp.float32)
        + b_ref[:, :]
    )


def _linear(x, Wp, bp, block_rows):
    M, K = x.shape
    N = Wp.shape[1]
    assert M % block_rows == 0
    return pl.pallas_call(
        _linear_body,
        grid=(M // block_rows,),
        in_specs=[
            pl.BlockSpec((block_rows, K), lambda i: (i, 0)),
            pl.BlockSpec((K, N), lambda i: (0, 0)),
            pl.BlockSpec((1, N), lambda i: (0, 0)),
        ],
        out_specs=pl.BlockSpec((block_rows, N), lambda i: (i, 0)),
        out_shape=jax.ShapeDtypeStruct((M, N), jnp.float32),
    )(x, Wp, bp)


def _node_update_body(h_ref, agg_ref, w1_ref, b1_ref, w2_ref, b2_ref,
                      g_ref, be_ref, o_ref):
    h = h_ref[:, :]
    c = h + agg_ref[0] + agg_ref[1]
    c = jnp.maximum(
        jnp.dot(c, w1_ref[:, :], preferred_element_type=jnp.float32) + b1_ref[:, :],
        0.0,
    )
    c = jnp.dot(c, w2_ref[:, :], preferred_element_type=jnp.float32) + b2_ref[:, :]
    n = c.shape[0]
    mean = jnp.sum(c, axis=0, keepdims=True) * (1.0 / n)
    d = c - mean
    var = jnp.sum(d * d, axis=0, keepdims=True) * (1.0 / n)
    c = d * jax.lax.rsqrt(var + 1e-5) * g_ref[:, :] + be_ref[:, :]
    o_ref[:, :] = (h + jnp.maximum(c, 0.0)) * 0.5


def _node_update(h, agg2, lp):
    N = h.shape[0]
    w1, b1 = _pad_w(lp["conv1"], P, P)
    w2, b2 = _pad_w(lp["conv2"], P, P)
    g = jnp.zeros((1, P), jnp.float32).at[0, :H].set(lp["bn_gamma"])
    be = jnp.zeros((1, P), jnp.float32).at[0, :H].set(lp["bn_beta"])
    return pl.pallas_call(
        _node_update_body,
        in_specs=[pl.BlockSpec(h.shape, lambda: (0, 0)),
                  pl.BlockSpec(agg2.shape, lambda: (0, 0, 0)),
                  pl.BlockSpec((P, P), lambda: (0, 0)),
                  pl.BlockSpec((1, P), lambda: (0, 0)),
                  pl.BlockSpec((P, P), lambda: (0, 0)),
                  pl.BlockSpec((1, P), lambda: (0, 0)),
                  pl.BlockSpec((1, P), lambda: (0, 0)),
                  pl.BlockSpec((1, P), lambda: (0, 0))],
        out_specs=pl.BlockSpec(h.shape, lambda: (0, 0)),
        out_shape=jax.ShapeDtypeStruct(h.shape, jnp.float32),
    )(h, agg2, w1, b1, w2, b2, g, be)


def _edge_mlp_body(ts_ref, td_ref, tea_ref, w1a, w1b, w1c, b1, w2, b2, o_ref):
    tea = tea_ref[:, :]
    t = (jnp.dot(ts_ref[:, :], w1a[:, :], preferred_element_type=jnp.float32)
         + jnp.dot(td_ref[:, :], w1b[:, :], preferred_element_type=jnp.float32)
         + jnp.dot(tea, w1c[:, :], preferred_element_type=jnp.float32)
         + b1[:, :])
    t = jnp.maximum(t, 0.0)
    t = jnp.dot(t, w2[:, :], preferred_element_type=jnp.float32) + b2[:, :]
    o_ref[:, :] = tea + t * 0.5


def _edge_mlp(g_ts, g_td, tea, lp, block_rows):
    M = tea.shape[0]
    W1, b1v = lp["emlp1"]["W"], lp["emlp1"]["b"]
    w1a, _ = _pad_w({"W": W1[:H], "b": b1v}, P, P)
    w1b, _ = _pad_w({"W": W1[H:2 * H], "b": b1v}, P, P)
    w1c, b1 = _pad_w({"W": W1[2 * H:], "b": b1v}, P, P)
    w2, b2 = _pad_w(lp["emlp2"], P, P)
    row = lambda i: (i, 0)
    full = lambda i: (0, 0)
    return pl.pallas_call(
        _edge_mlp_body,
        grid=(M // block_rows,),
        in_specs=[pl.BlockSpec((block_rows, P), row),
                  pl.BlockSpec((block_rows, P), row),
                  pl.BlockSpec((block_rows, P), row),
                  pl.BlockSpec((P, P), full),
                  pl.BlockSpec((P, P), full),
                  pl.BlockSpec((P, P), full),
                  pl.BlockSpec((1, P), full),
                  pl.BlockSpec((P, P), full),
                  pl.BlockSpec((1, P), full)],
        out_specs=pl.BlockSpec((block_rows, P), row),
        out_shape=jax.ShapeDtypeStruct((M, P), jnp.float32),
    )(g_ts, g_td, tea, w1a, w1b, w1c, b1, w2, b2)


def _final_body(ts_ref, td_ref, tea_ref, w1a, w1b, w1c, b1, w2, b2, w3, b3,
                o_ref):
    f = (jnp.dot(jnp.maximum(ts_ref[:, :], 0.0), w1a[:, :],
                 preferred_element_type=jnp.float32)
         + jnp.dot(jnp.maximum(td_ref[:, :], 0.0), w1b[:, :],
                   preferred_element_type=jnp.float32)
         + jnp.dot(tea_ref[:, :], w1c[:, :], preferred_element_type=jnp.float32)
         + b1[:, :])
    f = jnp.maximum(f, 0.0)
    f = jnp.maximum(
        jnp.dot(f, w2[:, :], preferred_element_type=jnp.float32) + b2[:, :], 0.0)
    o_ref[:, :] = jnp.dot(f, w3[:, :], preferred_element_type=jnp.float32) + b3[:, :]


def _final(g_ts, g_td, tea, m, block_rows):
    M = tea.shape[0]
    W1, b1v = m["l1"]["W"], m["l1"]["b"]
    D1 = W1.shape[1]
    w1a, _ = _pad_w({"W": W1[:H], "b": b1v}, P, D1)
    w1b, _ = _pad_w({"W": W1[H:2 * H], "b": b1v}, P, D1)
    w1c, b1 = _pad_w({"W": W1[2 * H:], "b": b1v}, P, D1)
    w2, b2 = m["l2"]["W"], m["l2"]["b"].reshape(1, -1)
    w3, b3 = m["l3"]["W"], m["l3"]["b"].reshape(1, -1)
    D2, D3 = w2.shape[1], w3.shape[1]
    row = lambda i: (i, 0)
    full = lambda i: (0, 0)
    return pl.pallas_call(
        _final_body,
        grid=(M // block_rows,),
        in_specs=[pl.BlockSpec((block_rows, P), row),
                  pl.BlockSpec((block_rows, P), row),
                  pl.BlockSpec((block_rows, P), row),
                  pl.BlockSpec((P, D1), full),
                  pl.BlockSpec((P, D1), full),
                  pl.BlockSpec((P, D1), full),
                  pl.BlockSpec((1, D1), full),
                  pl.BlockSpec((D1, D2), full),
                  pl.BlockSpec((1, D2), full),
                  pl.BlockSpec((D2, D3), full),
                  pl.BlockSpec((1, D3), full)],
        out_specs=pl.BlockSpec((block_rows, D3), row),
        out_shape=jax.ShapeDtypeStruct((M, D3), jnp.float32),
    )(g_ts, g_td, tea, w1a, w1b, w1c, b1, w2, b2, w3, b3)


def kernel(x, edge_index, edge_attr, edge_label_index, target_edge_attr, params):
    src = edge_index[0]
    dst = edge_index[1]
    ts = edge_label_index[0]
    td = edge_label_index[1]
    n_nodes = x.shape[0]

    wn, bn = _pad_w(params["node_emb"], x.shape[1], P)
    we, be = _pad_w(params["edge_emb"], edge_attr.shape[1], P)

    h = _linear(x, wn, bn, 2000)
    ea = _linear(edge_attr, we, be, 8000)
    tea = _linear(target_edge_attr, we, be, 8192)

    for lp in params["layers"]:
        msg = jnp.maximum(h[src] + ea, 0.0)
        aggr = jax.ops.segment_sum(msg, dst, num_segments=n_nodes)
        agg2 = jnp.stack([aggr, jnp.zeros_like(aggr)])
        h = _node_update(h, agg2, lp)
        g_ts = h[ts]
        g_td = h[td]
        tea = _edge_mlp(g_ts, g_td, tea, lp, 8192)

    return _final(g_ts, g_td, tea, params["mlp"], 8192)


# speedup vs baseline: 2.0087x; 1.9363x over previous
import functools

import jax
import jax.numpy as jnp
from jax import lax
from jax.experimental import pallas as pl
from jax.experimental.pallas import tpu as pltpu
from jax.experimental.pallas import tpu_sc as plsc

H = 66
P = 128
PH = 128

NC = 2
NS = 16
NW = NC * NS

N_EDGES = 320000
E_PER_TILE = N_EDGES // NW
CHUNK = 80
N_CHUNKS = E_PER_TILE // CHUNK

N_TGT = 65536
T_PER_TILE = N_TGT // NW
T_CHUNK = 128
T_CHUNKS = T_PER_TILE // T_CHUNK


def _pad_w(Wb, pin, pout):
    W, b = Wb["W"], Wb["b"]
    Wp = jnp.zeros((pin, pout), jnp.float32).at[: W.shape[0], : W.shape[1]].set(W)
    bp = jnp.zeros((1, pout), jnp.float32).at[0, : b.shape[0]].set(b)
    return Wp, bp


def _linear_body(x_ref, w_ref, b_ref, o_ref):
    o_ref[:, :] = (
        jnp.dot(x_ref[:, :], w_ref[:, :], preferred_element_type=jnp.float32)
        + b_ref[:, :]
    )


def _linear(x, Wp, bp, block_rows):
    M, K = x.shape
    N = Wp.shape[1]
    assert M % block_rows == 0
    return pl.pallas_call(
        _linear_body,
        grid=(M // block_rows,),
        in_specs=[
            pl.BlockSpec((block_rows, K), lambda i: (i, 0)),
            pl.BlockSpec((K, N), lambda i: (0, 0)),
            pl.BlockSpec((1, N), lambda i: (0, 0)),
        ],
        out_specs=pl.BlockSpec((block_rows, N), lambda i: (i, 0)),
        out_shape=jax.ShapeDtypeStruct((M, N), jnp.float32),
    )(x, Wp, bp)


def _node_update_body(h_ref, agg_ref, w1_ref, b1_ref, w2_ref, b2_ref,
                      g_ref, be_ref, o_ref):
    h = h_ref[:, :]
    c = h[:, :P] + agg_ref[0] + agg_ref[1]
    c = jnp.maximum(
        jnp.dot(c, w1_ref[:, :], preferred_element_type=jnp.float32) + b1_ref[:, :],
        0.0,
    )
    c = jnp.dot(c, w2_ref[:, :], preferred_element_type=jnp.float32) + b2_ref[:, :]
    n = c.shape[0]
    mean = jnp.sum(c, axis=0, keepdims=True) * (1.0 / n)
    d = c - mean
    var = jnp.sum(d * d, axis=0, keepdims=True) * (1.0 / n)
    c = d * jax.lax.rsqrt(var + 1e-5) * g_ref[:, :] + be_ref[:, :]
    r = (h[:, :P] + jnp.maximum(c, 0.0)) * 0.5
    if PH > P:
        r = jnp.concatenate([r, jnp.zeros((n, PH - P), jnp.float32)], axis=1)
    o_ref[:, :] = r


def _node_update(h, agg2, lp):
    w1, b1 = _pad_w(lp["conv1"], P, P)
    w2, b2 = _pad_w(lp["conv2"], P, P)
    g = jnp.zeros((1, P), jnp.float32).at[0, :H].set(lp["bn_gamma"])
    be = jnp.zeros((1, P), jnp.float32).at[0, :H].set(lp["bn_beta"])
    return pl.pallas_call(
        _node_update_body,
        in_specs=[pl.BlockSpec(h.shape, lambda: (0, 0)),
                  pl.BlockSpec(agg2.shape, lambda: (0, 0, 0)),
                  pl.BlockSpec((P, P), lambda: (0, 0)),
                  pl.BlockSpec((1, P), lambda: (0, 0)),
                  pl.BlockSpec((P, P), lambda: (0, 0)),
                  pl.BlockSpec((1, P), lambda: (0, 0)),
                  pl.BlockSpec((1, P), lambda: (0, 0)),
                  pl.BlockSpec((1, P), lambda: (0, 0))],
        out_specs=pl.BlockSpec(h.shape, lambda: (0, 0)),
        out_shape=jax.ShapeDtypeStruct(h.shape, jnp.float32),
    )(h, agg2, w1, b1, w2, b2, g, be)


def _edge_mlp_body(ts_ref, td_ref, tea_ref, w1a, w1b, w1c, b1, w2, b2, o_ref):
    tea = tea_ref[:, :]
    t = (jnp.dot(ts_ref[:, :], w1a[:, :], preferred_element_type=jnp.float32)
         + jnp.dot(td_ref[:, :], w1b[:, :], preferred_element_type=jnp.float32)
         + jnp.dot(tea, w1c[:, :], preferred_element_type=jnp.float32)
         + b1[:, :])
    t = jnp.maximum(t, 0.0)
    t = jnp.dot(t, w2[:, :], preferred_element_type=jnp.float32) + b2[:, :]
    o_ref[:, :] = tea + t * 0.5


def _edge_mlp(g_ts, g_td, tea, lp, block_rows):
    M = tea.shape[0]
    W1, b1v = lp["emlp1"]["W"], lp["emlp1"]["b"]
    w1a, _ = _pad_w({"W": W1[:H], "b": b1v}, PH, P)
    w1b, _ = _pad_w({"W": W1[H:2 * H], "b": b1v}, PH, P)
    w1c, b1 = _pad_w({"W": W1[2 * H:], "b": b1v}, P, P)
    w2, b2 = _pad_w(lp["emlp2"], P, P)
    row = lambda i: (i, 0)
    full = lambda i: (0, 0)
    return pl.pallas_call(
        _edge_mlp_body,
        grid=(M // block_rows,),
        in_specs=[pl.BlockSpec((block_rows, PH), row),
                  pl.BlockSpec((block_rows, PH), row),
                  pl.BlockSpec((block_rows, P), row),
                  pl.BlockSpec((PH, P), full),
                  pl.BlockSpec((PH, P), full),
                  pl.BlockSpec((P, P), full),
                  pl.BlockSpec((1, P), full),
                  pl.BlockSpec((P, P), full),
                  pl.BlockSpec((1, P), full)],
        out_specs=pl.BlockSpec((block_rows, P), row),
        out_shape=jax.ShapeDtypeStruct((M, P), jnp.float32),
    )(g_ts, g_td, tea, w1a, w1b, w1c, b1, w2, b2)


def _final_body(ts_ref, td_ref, tea_ref, w1a, w1b, w1c, b1, w2, b2, w3, b3,
                o_ref):
    f = (jnp.dot(jnp.maximum(ts_ref[:, :], 0.0), w1a[:, :],
                 preferred_element_type=jnp.float32)
         + jnp.dot(jnp.maximum(td_ref[:, :], 0.0), w1b[:, :],
                   preferred_element_type=jnp.float32)
         + jnp.dot(tea_ref[:, :], w1c[:, :], preferred_element_type=jnp.float32)
         + b1[:, :])
    f = jnp.maximum(f, 0.0)
    f = jnp.maximum(
        jnp.dot(f, w2[:, :], preferred_element_type=jnp.float32) + b2[:, :], 0.0)
    o_ref[:, :] = jnp.dot(f, w3[:, :], preferred_element_type=jnp.float32) + b3[:, :]


def _final(g_ts, g_td, tea, m, block_rows):
    M = tea.shape[0]
    W1, b1v = m["l1"]["W"], m["l1"]["b"]
    D1 = W1.shape[1]
    w1a, _ = _pad_w({"W": W1[:H], "b": b1v}, PH, D1)
    w1b, _ = _pad_w({"W": W1[H:2 * H], "b": b1v}, PH, D1)
    w1c, b1 = _pad_w({"W": W1[2 * H:], "b": b1v}, P, D1)
    w2, b2 = m["l2"]["W"], m["l2"]["b"].reshape(1, -1)
    w3, b3 = m["l3"]["W"], m["l3"]["b"].reshape(1, -1)
    D2, D3 = w2.shape[1], w3.shape[1]
    row = lambda i: (i, 0)
    full = lambda i: (0, 0)
    return pl.pallas_call(
        _final_body,
        grid=(M // block_rows,),
        in_specs=[pl.BlockSpec((block_rows, PH), row),
                  pl.BlockSpec((block_rows, PH), row),
                  pl.BlockSpec((block_rows, P), row),
                  pl.BlockSpec((PH, D1), full),
                  pl.BlockSpec((PH, D1), full),
                  pl.BlockSpec((P, D1), full),
                  pl.BlockSpec((1, D1), full),
                  pl.BlockSpec((D1, D2), full),
                  pl.BlockSpec((1, D2), full),
                  pl.BlockSpec((D2, D3), full),
                  pl.BlockSpec((1, D3), full)],
        out_specs=pl.BlockSpec((block_rows, D3), row),
        out_shape=jax.ShapeDtypeStruct((M, D3), jnp.float32),
    )(g_ts, g_td, tea, w1a, w1b, w1c, b1, w2, b2, w3, b3)


def _rank_body(d_ref, o_ref):
    d = d_ref[:, :]
    m, w = d.shape
    lane = jax.lax.broadcasted_iota(jnp.int32, (m, w), 1)
    rank = jnp.zeros((m, w), jnp.int32)
    for k in range(w):
        eq = d == d[:, k:k + 1]
        rank = rank + jnp.where(jnp.logical_and(eq, lane > k), 1, 0)
    o_ref[:, :] = rank


def _edge_rank(dst2):
    M = dst2.shape[0]
    blk = 1000
    return pl.pallas_call(
        _rank_body,
        grid=(M // blk,),
        in_specs=[pl.BlockSpec((blk, CHUNK), lambda i: (i, 0))],
        out_specs=pl.BlockSpec((blk, CHUNK), lambda i: (i, 0)),
        out_shape=jax.ShapeDtypeStruct((M, CHUNK), jnp.int32),
    )(dst2)


DUMP = 10000


def _sc_edge_aggr(h, ea, src3, dst3, rnk3):
    n = h.shape[0]
    rows_per_tile = 640
    last_r0 = n - rows_per_tile
    nz = rows_per_tile // CHUNK
    ng = CHUNK // 16
    mesh = plsc.VectorSubcoreMesh(core_axis_name="c", subcore_axis_name="s")

    @functools.partial(
        pl.kernel,
        out_type=jax.ShapeDtypeStruct((NC, n, P), jnp.float32),
        mesh=mesh,
        scratch_types=[
            pltpu.VMEM((CHUNK,), jnp.int32),
            pltpu.VMEM((CHUNK,), jnp.int32),
            pltpu.VMEM((CHUNK,), jnp.int32),
            pltpu.VMEM((CHUNK, P), jnp.float32),
            pltpu.VMEM((CHUNK, PH), jnp.float32),
            pltpu.VMEM_SHARED((n + 8, PH), jnp.float32),
            pltpu.SemaphoreType.DMA,
        ],
    )
    def k(h_hbm, ea_hbm, src_hbm, dst_hbm, rnk_hbm, out_hbm,
          sidx_c, didx_c, ridx_c, bea, bh, acc, sem):
        cid = lax.axis_index("c")
        sid = lax.axis_index("s")
        wid = sid * NC + cid

        def zrow(i, _):
            for j in range(P // 16):
                bea[i, pl.ds(j * 16, 16)] = jnp.zeros((16,), jnp.float32)
            return 0
        lax.fori_loop(0, CHUNK, zrow, 0)
        r0 = jnp.where(sid == NS - 1, last_r0, sid * rows_per_tile)
        for z in range(nz):
            pltpu.sync_copy(bea, acc.at[pl.ds(r0 + z * CHUNK, CHUNK), :])
        plsc.subcore_barrier()

        def relu_row(i, _):
            for j in range(P // 16):
                s = pl.ds(j * 16, 16)
                bea[i, s] = jnp.maximum(bea[i, s] + bh[i, s], 0.0)
            return 0

        NR = 3

        def chunk_body(c, _):
            base = wid * E_PER_TILE + c * CHUNK
            pltpu.sync_copy(src_hbm.at[wid, c], sidx_c)
            pltpu.sync_copy(dst_hbm.at[wid, c], didx_c)
            pltpu.sync_copy(rnk_hbm.at[wid, c], ridx_c)
            pltpu.sync_copy(ea_hbm.at[pl.ds(base, CHUNK), :], bea)
            pltpu.async_copy(h_hbm.at[sidx_c], bh, sem).wait()
            lax.fori_loop(0, CHUNK, relu_row, 0)

            for r in range(NR):
                for g in range(ng):
                    d16 = didx_c[pl.ds(g * 16, 16)]
                    r16 = ridx_c[pl.ds(g * 16, 16)]
                    sidx_c[pl.ds(g * 16, 16)] = jnp.where(
                        r16 == jnp.full((16,), r, jnp.int32),
                        d16, jnp.full((16,), DUMP, jnp.int32))
                pltpu.sync_copy(bea, acc.at[sidx_c], add=True)
            return 0

        lax.fori_loop(0, N_CHUNKS, chunk_body, 0)

        plsc.subcore_barrier()
        pltpu.sync_copy(acc.at[pl.ds(r0, rows_per_tile), :],
                        out_hbm.at[cid, pl.ds(r0, rows_per_tile), :])

    return k(h, ea, src3, dst3, rnk3)


def _sc_gather_pairs(h, idx4):
    mesh = plsc.VectorSubcoreMesh(core_axis_name="c", subcore_axis_name="s")

    @functools.partial(
        pl.kernel,
        out_type=jax.ShapeDtypeStruct((2, N_TGT, PH), jnp.float32),
        mesh=mesh,
        scratch_types=[
            pltpu.VMEM((T_CHUNKS, T_CHUNK), jnp.int32),
            pltpu.VMEM((T_CHUNK, PH), jnp.float32),
            pltpu.SemaphoreType.DMA,
        ],
    )
    def k(h_hbm, idx_hbm, out_hbm, idx_v, buf, sem):
        cid = lax.axis_index("c")
        sid = lax.axis_index("s")
        wid = sid * NC + cid
        for side in range(2):
            pltpu.sync_copy(idx_hbm.at[side, wid], idx_v)

            def cb(c, _):
                pltpu.async_copy(h_hbm.at[idx_v.at[c]], buf, sem).wait()
                pltpu.sync_copy(
                    buf,
                    out_hbm.at[side,
                               pl.ds(wid * T_PER_TILE + c * T_CHUNK, T_CHUNK),
                               :])
                return 0
            lax.fori_loop(0, T_CHUNKS, cb, 0)

    return k(h, idx4)


def kernel(x, edge_index, edge_attr, edge_label_index, target_edge_attr, params):
    src = edge_index[0]
    dst = edge_index[1]
    ts = edge_label_index[0]
    td = edge_label_index[1]
    n_nodes = x.shape[0]

    wn, bn = _pad_w(params["node_emb"], x.shape[1], PH)
    we, be = _pad_w(params["edge_emb"], edge_attr.shape[1], P)

    h = _linear(x, wn, bn, 2000)
    ea = _linear(edge_attr, we, be, 8000)
    tea = _linear(target_edge_attr, we, be, 8192)

    src3 = src.reshape(NW, N_CHUNKS, CHUNK)
    dst3 = dst.reshape(NW, N_CHUNKS, CHUNK)
    rnk3 = _edge_rank(dst.reshape(NW * N_CHUNKS, CHUNK)).reshape(
        NW, N_CHUNKS, CHUNK)
    idx4 = jnp.stack([ts, td]).reshape(2, NW, T_CHUNKS, T_CHUNK)

    for lp in params["layers"]:
        agg2 = _sc_edge_aggr(h, ea, src3, dst3, rnk3)
        h = _node_update(h, agg2, lp)
        g = _sc_gather_pairs(h, idx4)
        g_ts, g_td = g[0], g[1]
        tea = _edge_mlp(g_ts, g_td, tea, lp, 8192)

    return _final(g_ts, g_td, tea, params["mlp"], 8192)
